# baseline (speedup 1.0000x reference)
import functools

import jax
import jax.numpy as jnp
import numpy as np
from jax import lax
from jax.experimental import pallas as pl
from jax.experimental.pallas import tpu as pltpu
from jax.experimental.pallas import tpu_sc as plsc

N = 10000
E = 320000
D = 128
DH = D // 2
NC = 2
NS = 16
EPT = E // NS
CH = 80
NCHUNK = EPT // CH
RPT = 624
RTAIL = N - NS * RPT

_PERM = np.concatenate([
    np.stack([np.arange(0, 16), np.arange(32, 48)], 1).ravel(),
    np.stack([np.arange(16, 32), np.arange(48, 64)], 1).ravel(),
])

_mesh = plsc.VectorSubcoreMesh(core_axis_name="c", subcore_axis_name="s")


@functools.partial(
    pl.kernel,
    mesh=_mesh,
    compiler_params=pltpu.CompilerParams(
        use_tc_tiling_on_sc=False, needs_layout_passes=False),
    out_type=jax.ShapeDtypeStruct((NC, N, DH), jnp.float32),
    scratch_types=[
        pltpu.VMEM((NCHUNK + 2, CH), jnp.int32),
        pltpu.VMEM((NCHUNK, CH), jnp.int32),
        pltpu.VMEM((CH, DH // 2), jnp.int32),
        pltpu.VMEM((CH, DH // 2), jnp.int32),
        pltpu.VMEM((CH, DH), jnp.float32),
        pltpu.VMEM((CH, DH), jnp.float32),
        pltpu.VMEM((CH,), jnp.float32),
        pltpu.VMEM((CH,), jnp.float32),
        pltpu.VMEM_SHARED((N, DH // 2), jnp.int32),
        pltpu.VMEM_SHARED((N, DH), jnp.float32),
        pltpu.SemaphoreType.DMA,
        pltpu.SemaphoreType.DMA,
        pltpu.SemaphoreType.DMA,
        pltpu.SemaphoreType.DMA,
        pltpu.SemaphoreType.DMA,
        pltpu.SemaphoreType.DMA,
    ],
)
def _aggregate(xs_hbm, sd_hbm, ew_hbm, out_hbm,
               src_v, dst_v, g0_v, g1_v, s0_v, s1_v, ew0_v, ew1_v,
               xsp, acc, sem0, sem1, ssem0, ssem1, wsem0, wsem1):
    cc = lax.axis_index("c")
    ss = lax.axis_index("s")

    pltpu.sync_copy(sd_hbm.at[ss], src_v.at[pl.ds(0, NCHUNK)])
    pltpu.sync_copy(xs_hbm.at[cc, pl.ds(ss * RPT, RPT)],
                    xsp.at[pl.ds(ss * RPT, RPT)])

    @pl.when(ss == NS - 1)
    def _stage_tail():
        pltpu.sync_copy(xs_hbm.at[cc, pl.ds(NS * RPT, RTAIL)],
                        xsp.at[pl.ds(NS * RPT, RTAIL)])

    lomask = jnp.int32((1 << 14) - 1)

    def _unpk(i, _):
        for j in range(CH // 16):
            sl = pl.ds(j * 16, 16)
            v = src_v[i, sl]
            dst_v[i, sl] = v >> 14
            src_v[i, sl] = v & lomask
        return 0
    lax.fori_loop(0, NCHUNK, _unpk, 0)

    def _zpad(i, _):
        for j in range(CH // 16):
            src_v[NCHUNK + i, pl.ds(j * 16, 16)] = jnp.zeros((16,), jnp.int32)
        return 0
    lax.fori_loop(0, 2, _zpad, 0)

    def _zrow(i, _):
        for j in range(DH // 16):
            s0_v[i, pl.ds(j * 16, 16)] = jnp.zeros((16,), jnp.float32)
            s1_v[i, pl.ds(j * 16, 16)] = jnp.zeros((16,), jnp.float32)
        return 0
    lax.fori_loop(0, CH, _zrow, 0)
    for k in range(7):
        pltpu.sync_copy(s0_v, acc.at[pl.ds(ss * RPT + k * CH, CH)])
    pltpu.sync_copy(s0_v.at[pl.ds(0, RPT - 7 * CH)],
                    acc.at[pl.ds(ss * RPT + 7 * CH, RPT - 7 * CH)])

    @pl.when(ss == NS - 1)
    def _zero_tail():
        pltpu.sync_copy(s0_v.at[pl.ds(0, RTAIL)],
                        acc.at[pl.ds(NS * RPT, RTAIL)])

    plsc.subcore_barrier()

    himask = jnp.int32(-65536)

    def _scale(gbuf, sbuf, ewb):
        def _grp(g, _):
            wvec = ewb[pl.ds(g * 16, 16)]
            for l in range(16):
                e = g * 16 + l
                w = wvec[l]
                for h in range(2):
                    v = gbuf[e, pl.ds(h * 16, 16)]
                    lo = plsc.bitcast(v << 16, jnp.float32)
                    hi = plsc.bitcast(v & himask, jnp.float32)
                    sbuf[e, pl.ds(h * 16, 16)] = lo * w
                    sbuf[e, pl.ds(h * 16 + 32, 16)] = hi * w
            return 0
        lax.fori_loop(0, CH // 16, _grp, 0)

    pltpu.async_copy(xsp.at[src_v.at[0]], g0_v, sem0)
    pltpu.async_copy(xsp.at[src_v.at[1]], g1_v, sem1)
    pltpu.async_copy(ew_hbm.at[ss, 0], ew0_v, wsem0)
    pltpu.async_copy(ew_hbm.at[ss, 1], ew1_v, wsem1)
    pltpu.async_copy(s0_v, acc.at[dst_v.at[0]], ssem0, add=True)
    pltpu.async_copy(s1_v, acc.at[dst_v.at[1]], ssem1, add=True)

    def _pair(i, _):
        c0 = i * 2
        pltpu.make_async_copy(xsp.at[src_v.at[c0]], g0_v, sem0).wait()
        pltpu.make_async_copy(ew_hbm.at[ss, c0], ew0_v, wsem0).wait()
        pltpu.make_async_copy(s0_v, acc.at[dst_v.at[c0]], ssem0).wait()
        _scale(g0_v, s0_v, ew0_v)
        pltpu.async_copy(xsp.at[src_v.at[c0 + 2]], g0_v, sem0)
        pltpu.async_copy(ew_hbm.at[ss, c0 + 2], ew0_v, wsem0)
        pltpu.async_copy(s0_v, acc.at[dst_v.at[c0]], ssem0, add=True)

        pltpu.make_async_copy(xsp.at[src_v.at[c0 + 1]], g1_v, sem1).wait()
        pltpu.make_async_copy(ew_hbm.at[ss, c0 + 1], ew1_v, wsem1).wait()
        pltpu.make_async_copy(s1_v, acc.at[dst_v.at[c0 + 1]], ssem1).wait()
        _scale(g1_v, s1_v, ew1_v)
        pltpu.async_copy(xsp.at[src_v.at[c0 + 3]], g1_v, sem1)
        pltpu.async_copy(ew_hbm.at[ss, c0 + 3], ew1_v, wsem1)
        pltpu.async_copy(s1_v, acc.at[dst_v.at[c0 + 1]], ssem1, add=True)
        return 0

    lax.fori_loop(0, NCHUNK // 2, _pair, 0)
    pltpu.make_async_copy(s0_v, acc.at[dst_v.at[0]], ssem0).wait()
    pltpu.make_async_copy(s1_v, acc.at[dst_v.at[1]], ssem1).wait()
    pltpu.make_async_copy(xsp.at[src_v.at[NCHUNK]], g0_v, sem0).wait()
    pltpu.make_async_copy(xsp.at[src_v.at[NCHUNK + 1]], g1_v, sem1).wait()
    pltpu.make_async_copy(ew_hbm.at[ss, NCHUNK], ew0_v, wsem0).wait()
    pltpu.make_async_copy(ew_hbm.at[ss, NCHUNK + 1], ew1_v, wsem1).wait()
    plsc.subcore_barrier()

    pltpu.sync_copy(acc.at[pl.ds(ss * RPT, RPT)],
                    out_hbm.at[cc, pl.ds(ss * RPT, RPT)])

    @pl.when(ss == NS - 1)
    def _write_tail():
        pltpu.sync_copy(acc.at[pl.ds(NS * RPT, RTAIL)],
                        out_hbm.at[cc, pl.ds(NS * RPT, RTAIL)])


_BM = 1000


def _mm_body(p_ref, w_ref, b_ref, o_ref):
    o_ref[...] = (
        jnp.dot(p_ref[0], w_ref[0], preferred_element_type=jnp.float32)
        + jnp.dot(p_ref[1], w_ref[1], preferred_element_type=jnp.float32)
        + b_ref[...]
    )


def _finish(partials, W2, b2):
    return pl.pallas_call(
        _mm_body,
        grid=(N // _BM,),
        in_specs=[
            pl.BlockSpec((NC, _BM, DH), lambda i: (0, i, 0)),
            pl.BlockSpec((NC, DH, D), lambda i: (0, 0, 0)),
            pl.BlockSpec((1, D), lambda i: (0, 0)),
        ],
        out_specs=pl.BlockSpec((_BM, D), lambda i: (i, 0)),
        out_shape=jax.ShapeDtypeStruct((N, D), jnp.float32),
    )(partials, W2, b2)


def kernel(X, edge_index, edge_weight, W, b):
    src = edge_index[0].astype(jnp.int32)
    dst = edge_index[1].astype(jnp.int32)
    sd = (src | (dst << 14)).reshape(NS, NCHUNK, CH)
    ew = jnp.pad(edge_weight.reshape(NS, NCHUNK, CH), ((0, 0), (0, 2), (0, 0)))
    xh = X.reshape(N, NC, DH).transpose(1, 0, 2)
    xbf = xh[:, :, _PERM].astype(jnp.bfloat16)
    xs = lax.bitcast_convert_type(
        xbf.reshape(NC, N, DH // 2, 2), jnp.int32)
    partials = _aggregate(xs, sd, ew)
    w2 = jnp.stack([W[:DH], W[DH:]])
    return _finish(partials, w2, b.reshape(1, D))
